# Initial kernel scaffold; baseline (speedup 1.0000x reference)
import jax
import jax.numpy as jnp
from jax.experimental import pallas as pl


def kernel(x, W_gate, W_up, W_down):
    raise NotImplementedError("write your pallas kernel here")


# speedup vs baseline: 2.6962x; 2.6962x over previous
import functools

import jax
import jax.numpy as jnp
from jax.experimental import pallas as pl
from jax.experimental.pallas import tpu as pltpu

_N_SEG = 8
_SEG = 1024
_D = 512
_H = 1024


def _ffn_body(x_ref, wg_ref, wu_ref, wd_ref, o_ref):
    xb = x_ref[...]
    g = jnp.dot(xb, wg_ref[0], preferred_element_type=jnp.float32)
    u = jnp.dot(xb, wu_ref[0], preferred_element_type=jnp.float32)
    h = (g * jax.nn.sigmoid(g)) * u
    o_ref[...] = jnp.dot(h, wd_ref[0], preferred_element_type=jnp.float32)


@jax.jit
def kernel(x, W_gate, W_up, W_down):
    grid = (_N_SEG,)
    out = pl.pallas_call(
        _ffn_body,
        grid=grid,
        in_specs=[
            pl.BlockSpec((_SEG, _D), lambda i: (i, 0)),
            pl.BlockSpec((1, _D, _H), lambda i: (i, 0, 0)),
            pl.BlockSpec((1, _D, _H), lambda i: (i, 0, 0)),
            pl.BlockSpec((1, _H, _D), lambda i: (i, 0, 0)),
        ],
        out_specs=pl.BlockSpec((_SEG, _D), lambda i: (i, 0)),
        out_shape=jax.ShapeDtypeStruct((_N_SEG * _SEG, _D), jnp.float32),
    )(x, W_gate, W_up, W_down)
    return out
